# Initial kernel scaffold; baseline (speedup 1.0000x reference)
import jax
import jax.numpy as jnp
from jax.experimental import pallas as pl


def kernel(x, table):
    raise NotImplementedError("write your pallas kernel here")


# speedup vs baseline: 2.0295x; 2.0295x over previous
import functools

import jax
import jax.numpy as jnp
from jax import lax
from jax.experimental import pallas as pl
from jax.experimental.pallas import tpu as pltpu
from jax.experimental.pallas import tpu_sc as plsc

D = 64
L = 16
MAX_NORM = 1.0
EPS = 1e-7
NC = 2
NS = 16
NW = NC * NS
CHUNK = 128


@functools.lru_cache(maxsize=None)
def _build(B):
    assert B % (NW * CHUNK) == 0
    nch = B // (NW * CHUNK)

    mesh = plsc.VectorSubcoreMesh(core_axis_name="c", subcore_axis_name="s")

    @functools.partial(
        pl.kernel,
        mesh=mesh,
        compiler_params=pltpu.CompilerParams(use_tc_tiling_on_sc=False),
        out_type=jax.ShapeDtypeStruct((B, D), jnp.float32),
        scratch_types=[
            pltpu.VMEM((nch, CHUNK), jnp.int32),
            pltpu.VMEM((CHUNK, D), jnp.float32),
            pltpu.SemaphoreType.DMA,
        ],
    )
    def k(table_hbm, idx_hbm, out_hbm, idx_v, rows_v, sem):
        wid = lax.axis_index("s") * NC + lax.axis_index("c")
        pltpu.sync_copy(idx_hbm.at[pl.ds(wid * nch, nch)], idx_v)

        lanes = lax.iota(jnp.int32, L)
        perms = [lanes ^ k for k in (1, 2, 4, 8)]

        def chunk_body(c, carry):
            pltpu.async_copy(table_hbm.at[idx_v.at[c]], rows_v, sem).wait()

            def row_body(r, carry2):
                a0 = rows_v[r, pl.ds(0, L)]
                a1 = rows_v[r, pl.ds(L, L)]
                a2 = rows_v[r, pl.ds(2 * L, L)]
                a3 = rows_v[r, pl.ds(3 * L, L)]
                n = jnp.abs(a0) + jnp.abs(a1) + jnp.abs(a2) + jnp.abs(a3)
                for p in perms:
                    n = n + n.at[p].get(mode="promise_in_bounds")
                s = jnp.where(n > MAX_NORM, MAX_NORM / (n + EPS), jnp.float32(1.0))
                rows_v[r, pl.ds(0, L)] = a0 * s
                rows_v[r, pl.ds(L, L)] = a1 * s
                rows_v[r, pl.ds(2 * L, L)] = a2 * s
                rows_v[r, pl.ds(3 * L, L)] = a3 * s
                return carry2

            lax.fori_loop(0, CHUNK, row_body, 0)
            pltpu.sync_copy(rows_v, out_hbm.at[pl.ds((wid * nch + c) * CHUNK, CHUNK)])
            return carry

        lax.fori_loop(0, nch, chunk_body, 0)

    return k


def kernel(x, table):
    B = x.size
    xf = x.reshape(B // CHUNK, CHUNK)
    out = _build(B)(table, xf)
    return out.reshape(x.shape + (D,))


# speedup vs baseline: 4.1008x; 2.0206x over previous
import functools

import jax
import jax.numpy as jnp
from jax import lax
from jax.experimental import pallas as pl
from jax.experimental.pallas import tpu as pltpu
from jax.experimental.pallas import tpu_sc as plsc

D = 64
L = 16
MAX_NORM = 1.0
EPS = 1e-7
NC = 2
NS = 16
NW = NC * NS
CHUNK = 128
NBUF = 2


@functools.lru_cache(maxsize=None)
def _build(B):
    assert B % (NW * CHUNK) == 0
    nch = B // (NW * CHUNK)
    assert nch % NBUF == 0

    mesh = plsc.VectorSubcoreMesh(core_axis_name="c", subcore_axis_name="s")

    @functools.partial(
        pl.kernel,
        mesh=mesh,
        compiler_params=pltpu.CompilerParams(use_tc_tiling_on_sc=False),
        out_type=jax.ShapeDtypeStruct((B, D), jnp.float32),
        scratch_types=[
            pltpu.VMEM((nch, CHUNK), jnp.int32),
            pltpu.VMEM((NBUF, CHUNK, D), jnp.float32),
            pltpu.VMEM((NBUF, CHUNK, D), jnp.float32),
            pltpu.SemaphoreType.DMA((NBUF,)),
            pltpu.SemaphoreType.DMA((NBUF,)),
        ],
    )
    def k(table_hbm, idx_hbm, out_hbm, idx_v, gbuf, obuf, gsem, osem):
        wid = lax.axis_index("s") * NC + lax.axis_index("c")
        base = wid * nch
        pltpu.sync_copy(idx_hbm.at[pl.ds(base, nch)], idx_v)

        lanes = lax.iota(jnp.int32, L)
        perms = [lanes ^ p for p in (1, 2, 4, 8)]

        def gather(c, b):
            return pltpu.make_async_copy(
                table_hbm.at[idx_v.at[c]], gbuf.at[b], gsem.at[b])

        def putback(c, b):
            return pltpu.make_async_copy(
                obuf.at[b], out_hbm.at[pl.ds((base + c) * CHUNK, CHUNK)],
                osem.at[b])

        for b in range(NBUF):
            gather(b, b).start()

        @pl.loop(0, nch, step=NBUF)
        def _(c0):
            for b in range(NBUF):
                c = c0 + b
                gather(c, b).wait()

                @pl.when(c0 > 0)
                def _():
                    putback(c - NBUF, b).wait()

                gb = gbuf.at[b]
                ob = obuf.at[b]

                @plsc.parallel_loop(0, CHUNK, unroll=4)
                def _(r):
                    a0 = gb[r, pl.ds(0, L)]
                    a1 = gb[r, pl.ds(L, L)]
                    a2 = gb[r, pl.ds(2 * L, L)]
                    a3 = gb[r, pl.ds(3 * L, L)]
                    n = jnp.abs(a0) + jnp.abs(a1) + jnp.abs(a2) + jnp.abs(a3)
                    for p in perms:
                        n = n + n.at[p].get(mode="promise_in_bounds")
                    s = jnp.where(n > MAX_NORM, MAX_NORM / (n + EPS),
                                  jnp.float32(1.0))
                    ob[r, pl.ds(0, L)] = a0 * s
                    ob[r, pl.ds(L, L)] = a1 * s
                    ob[r, pl.ds(2 * L, L)] = a2 * s
                    ob[r, pl.ds(3 * L, L)] = a3 * s

                @pl.when(c + NBUF < nch)
                def _():
                    gather(c + NBUF, b).start()

                putback(c, b).start()

        for b in range(NBUF):
            putback(nch - NBUF + b, b).wait()

    return k


def kernel(x, table):
    B = x.size
    xf = x.reshape(B // CHUNK, CHUNK)
    out = _build(B)(table, xf)
    return out.reshape(x.shape + (D,))


# speedup vs baseline: 4.2226x; 1.0297x over previous
import functools

import jax
import jax.numpy as jnp
from jax import lax
from jax.experimental import pallas as pl
from jax.experimental.pallas import tpu as pltpu
from jax.experimental.pallas import tpu_sc as plsc

D = 64
L = 16
MAX_NORM = 1.0
EPS = 1e-7
NC = 2
NS = 16
NW = NC * NS
G0 = 128
NBUF = 2


@functools.lru_cache(maxsize=None)
def _build(NB, T):
    assert NB % NW == 0
    nb = NB // NW
    assert nb % NBUF == 0
    splits = [(0, G0), (G0, T - G0)] if T > G0 else [(0, T)]

    mesh = plsc.VectorSubcoreMesh(core_axis_name="c", subcore_axis_name="s")

    @functools.partial(
        pl.kernel,
        mesh=mesh,
        compiler_params=pltpu.CompilerParams(use_tc_tiling_on_sc=False),
        out_type=jax.ShapeDtypeStruct((NB, T, D), jnp.float32),
        scratch_types=[
            pltpu.VMEM((nb, T), jnp.int32),
            pltpu.VMEM((NBUF, T, D), jnp.float32),
            pltpu.VMEM((NBUF, T, D), jnp.float32),
            pltpu.SemaphoreType.DMA((NBUF,)),
            pltpu.SemaphoreType.DMA((NBUF,)),
        ],
    )
    def k(table_hbm, idx_hbm, out_hbm, idx_v, gbuf, obuf, gsem, osem):
        wid = lax.axis_index("s") * NC + lax.axis_index("c")
        base = wid * nb
        pltpu.sync_copy(idx_hbm.at[pl.ds(base, nb)], idx_v)

        lanes = lax.iota(jnp.int32, L)
        perms = [lanes ^ p for p in (1, 2, 4, 8)]

        def gathers(c, b):
            return [
                pltpu.make_async_copy(
                    table_hbm.at[idx_v.at[c, pl.ds(off, num)]],
                    gbuf.at[b, pl.ds(off, num)], gsem.at[b])
                for off, num in splits
            ]

        def putback(c, b):
            return pltpu.make_async_copy(
                obuf.at[b], out_hbm.at[base + c], osem.at[b])

        for b in range(NBUF):
            for g in gathers(b, b):
                g.start()

        @pl.loop(0, nb, step=NBUF)
        def _(c0):
            for b in range(NBUF):
                c = c0 + b
                for g in gathers(c, b):
                    g.wait()

                @pl.when(c0 > 0)
                def _():
                    putback(c - NBUF, b).wait()

                gb = gbuf.at[b]
                ob = obuf.at[b]

                @plsc.parallel_loop(0, T, unroll=4)
                def _(r):
                    a0 = gb[r, pl.ds(0, L)]
                    a1 = gb[r, pl.ds(L, L)]
                    a2 = gb[r, pl.ds(2 * L, L)]
                    a3 = gb[r, pl.ds(3 * L, L)]
                    n = jnp.abs(a0) + jnp.abs(a1) + jnp.abs(a2) + jnp.abs(a3)
                    for p in perms:
                        n = n + n.at[p].get(mode="promise_in_bounds")
                    s = jnp.where(n > MAX_NORM, MAX_NORM / (n + EPS),
                                  jnp.float32(1.0))
                    ob[r, pl.ds(0, L)] = a0 * s
                    ob[r, pl.ds(L, L)] = a1 * s
                    ob[r, pl.ds(2 * L, L)] = a2 * s
                    ob[r, pl.ds(3 * L, L)] = a3 * s

                @pl.when(c + NBUF < nb)
                def _():
                    for g in gathers(c + NBUF, b):
                        g.start()

                putback(c, b).start()

        for b in range(NBUF):
            putback(nb - NBUF + b, b).wait()

    return k


def kernel(x, table):
    NB, T = x.shape
    return _build(NB, T)(table, x)
